# baseline (speedup 1.0000x reference)
import functools

import jax
import jax.numpy as jnp
from jax import lax
from jax.experimental import pallas as pl
from jax.experimental.pallas import tpu as pltpu
from jax.experimental.pallas import tpu_sc as plsc

_NC = 2
_NS = 16
_NW = _NC * _NS
_LANES = 16


def _matvec_body(e_ref, w_ref, b_ref, t_ref):
    r = jax.lax.dot_general(
        w_ref[...], e_ref[...],
        (((1,), (1,)), ((), ())),
        preferred_element_type=jnp.float32,
    )
    t_ref[...] = r[0] + b_ref[0]


def _precompute_table(embed, W, b):
    V, D = embed.shape
    br = 1024
    return pl.pallas_call(
        _matvec_body,
        grid=(pl.cdiv(V, br),),
        in_specs=[
            pl.BlockSpec((br, D), lambda i: (i, 0)),
            pl.BlockSpec((1, D), lambda i: (0, 0)),
            pl.BlockSpec(memory_space=pltpu.SMEM),
        ],
        out_specs=pl.BlockSpec((br,), lambda i: (i,)),
        out_shape=jax.ShapeDtypeStruct((V,), jnp.float32),
    )(embed, W, b)


@functools.lru_cache(maxsize=None)
def _make_pool_kernel(B, L):
    n_rows = B // _NW
    n_idx = n_rows * L
    assert n_idx % 128 == 0 and n_rows % _LANES == 0
    n_chunks = n_idx // 128
    mesh = plsc.VectorSubcoreMesh(core_axis_name="c", subcore_axis_name="s")

    @functools.partial(
        pl.kernel,
        out_type=jax.ShapeDtypeStruct((B,), jnp.float32),
        mesh=mesh,
        scratch_types=[
            pltpu.VMEM((n_idx,), jnp.int32),
            pltpu.VMEM((n_chunks, 128), jnp.int32),
            pltpu.VMEM((n_idx,), jnp.float32),
            pltpu.VMEM((n_rows,), jnp.float32),
            pltpu.VMEM_SHARED((_NS * n_idx,), jnp.float32),
            pltpu.SemaphoreType.DMA,
            pltpu.SemaphoreType.DMA,
        ],
    )
    def pool(x_hbm, t_hbm, perm_hbm, out_hbm, idx_v, perm_v, vals_v, out_v,
             tr_sh, gsem, ssem):
        cid = lax.axis_index("c")
        sid = lax.axis_index("s")
        wid = sid * _NC + cid
        pltpu.sync_copy(x_hbm.at[pl.ds(wid * n_idx, n_idx)], idx_v)
        pltpu.sync_copy(perm_hbm.at[sid], perm_v)
        gathers = [
            pltpu.async_copy(
                t_hbm.at[idx_v.at[pl.ds(c * 128, 128)]],
                vals_v.at[pl.ds(c * 128, 128)],
                gsem,
            )
            for c in range(n_chunks)
        ]
        for cp in gathers:
            cp.wait()
        for c in range(n_chunks):
            pltpu.sync_copy(
                vals_v.at[pl.ds(c * 128, 128)],
                tr_sh.at[perm_v.at[c]],
            )
        pltpu.sync_copy(tr_sh.at[pl.ds(sid * n_idx, n_idx)], vals_v)
        inv_l = 1.0 / L
        for j in range(n_rows // _LANES):
            col = j * _LANES

            def body(l, acc):
                return acc + vals_v[pl.ds(l * n_rows + col, _LANES)]

            acc = lax.fori_loop(0, L, body, jnp.zeros((_LANES,), jnp.float32))
            out_v[pl.ds(col, _LANES)] = 1.0 / (1.0 + jnp.exp(acc * -inv_l))
        pltpu.sync_copy(out_v, out_hbm.at[pl.ds(wid * n_rows, n_rows)])

    return pool


def kernel(x, embed, W, b):
    B, L = x.shape
    n_rows = B // _NW
    n_idx = n_rows * L
    t = _precompute_table(embed, W.astype(jnp.float32), b.astype(jnp.float32))
    xf = x.astype(jnp.int32).reshape(B * L)
    e = jnp.arange(n_idx, dtype=jnp.int32)
    perm = (
        ((e % L) * n_rows + e // L)[None, :]
        + jnp.arange(_NS, dtype=jnp.int32)[:, None] * n_idx
    ).reshape(_NS, n_idx // 128, 128)
    out = _make_pool_kernel(B, L)(xf, t, perm)
    return out.reshape(B, 1)


# speedup vs baseline: 1.0925x; 1.0925x over previous
import functools
import math

import jax
import jax.numpy as jnp
from jax import lax
from jax.experimental import pallas as pl
from jax.experimental.pallas import tpu as pltpu
from jax.experimental.pallas import tpu_sc as plsc

_NC = 2
_NS = 16
_NW = _NC * _NS
_LANES = 16
_SPLIT = 4
_TBR = 1024


def _matvec_single(x_ref, e_ref, w_ref, b_ref, t_ref, xt_ref):
    r = jax.lax.dot_general(
        w_ref[...], e_ref[...],
        (((1,), (1,)), ((), ())),
        preferred_element_type=jnp.float32,
    )
    t_ref[...] = r[0] + b_ref[0]
    xt_ref[...] = jnp.transpose(x_ref[...])[None]


def _matvec_body(e0, e1, e2, e3, w_ref, b_ref, t_ref):
    qr = _TBR // _SPLIT
    for q, e_ref in enumerate((e0, e1, e2, e3)):
        r = jax.lax.dot_general(
            w_ref[...], e_ref[...],
            (((1,), (1,)), ((), ())),
            preferred_element_type=jnp.float32,
        )
        t_ref[pl.ds(q * qr, qr)] = r[0] + b_ref[0]


def _precompute_table(x, embed, W, b):
    V, D = embed.shape
    B, L = x.shape
    grid = pl.cdiv(V, _TBR)
    qr = _TBR // _SPLIT
    xbr = 512
    while B % xbr:
        xbr //= 2
    nb = B // xbr
    assert nb <= grid and xbr % 128 == 0

    def xin_map(i):
        return (jnp.minimum(i, nb - 1), 0)

    def xout_map(i):
        return (jnp.minimum(i, nb - 1), 0, 0)

    t, xt = pl.pallas_call(
        _matvec_single,
        grid=(grid,),
        in_specs=[
            pl.BlockSpec((xbr, L), xin_map),
            pl.BlockSpec((_TBR, D), lambda i: (i, 0)),
            pl.BlockSpec((1, D), lambda i: (0, 0)),
            pl.BlockSpec(memory_space=pltpu.SMEM),
        ],
        out_specs=[
            pl.BlockSpec((_TBR,), lambda i: (i,)),
            pl.BlockSpec((1, L, xbr), xout_map),
        ],
        out_shape=[
            jax.ShapeDtypeStruct((V,), jnp.float32),
            jax.ShapeDtypeStruct((nb, L, xbr), jnp.int32),
        ],
    )(x, embed, W, b)
    return t, xt


@functools.lru_cache(maxsize=None)
def _make_pool_kernel(B, L, xbr):
    n_rows = B // _NW
    n_idx = n_rows * L
    assert n_idx % 128 == 0 and n_rows % _LANES == 0 and n_rows == 128
    wpb = xbr // n_rows
    mesh = plsc.VectorSubcoreMesh(core_axis_name="c", subcore_axis_name="s")

    @functools.partial(
        pl.kernel,
        out_type=jax.ShapeDtypeStruct((B,), jnp.float32),
        mesh=mesh,
        scratch_types=[
            pltpu.VMEM((L, n_rows), jnp.int32),
            pltpu.VMEM((n_idx,), jnp.float32),
            pltpu.VMEM((n_rows,), jnp.float32),
            pltpu.SemaphoreType.DMA,
        ],
    )
    def pool(xt_hbm, t_hbm, out_hbm, idx_v, vals_v, out_v, sem):
        cid = lax.axis_index("c")
        sid = lax.axis_index("s")
        wid = sid * _NC + cid
        blk = wid // wpb
        off = (wid % wpb) * n_rows
        pltpu.sync_copy(xt_hbm.at[blk, :, pl.ds(off, n_rows)], idx_v)
        gathers = [
            pltpu.async_copy(
                t_hbm.at[idx_v.at[c]],
                vals_v.at[pl.ds(c * n_rows, n_rows)],
                sem,
            )
            for c in range(L)
        ]
        for cp in gathers:
            cp.wait()
        inv_l = 1.0 / L
        for j in range(n_rows // _LANES):
            col = j * _LANES

            def body(l, acc):
                return acc + vals_v[pl.ds(l * n_rows + col, _LANES)]

            acc = lax.fori_loop(0, L, body, jnp.zeros((_LANES,), jnp.float32))
            out_v[pl.ds(col, _LANES)] = 1.0 / (1.0 + jnp.exp(acc * -inv_l))
        pltpu.sync_copy(out_v, out_hbm.at[pl.ds(wid * n_rows, n_rows)])

    return pool


def kernel(x, embed, W, b):
    B, L = x.shape
    t, xt = _precompute_table(
        x.astype(jnp.int32), embed, W.astype(jnp.float32), b.astype(jnp.float32)
    )
    out = _make_pool_kernel(B, L, xt.shape[2])(xt, t)
    return out.reshape(B, 1)


# speedup vs baseline: 2.9937x; 2.7404x over previous
import functools

import jax
import jax.numpy as jnp
from jax import lax
from jax.experimental import pallas as pl
from jax.experimental.pallas import tpu as pltpu
from jax.experimental.pallas import tpu_sc as plsc

_NC = 2
_NS = 16
_NW = _NC * _NS
_LANES = 16
_TBC = 4096


def _matvec_body(et_ref, w_ref, b_ref, t_ref):
    r = jax.lax.dot_general(
        w_ref[...], et_ref[...],
        (((1,), (0,)), ((), ())),
        preferred_element_type=jnp.float32,
    )
    t_ref[...] = r[0] + b_ref[0]


def _precompute_table(embed_t, W, b):
    D, V = embed_t.shape
    return pl.pallas_call(
        _matvec_body,
        grid=(pl.cdiv(V, _TBC),),
        in_specs=[
            pl.BlockSpec((D, _TBC), lambda i: (0, i)),
            pl.BlockSpec((1, D), lambda i: (0, 0)),
            pl.BlockSpec(memory_space=pltpu.SMEM),
        ],
        out_specs=pl.BlockSpec((_TBC,), lambda i: (i,)),
        out_shape=jax.ShapeDtypeStruct((V,), jnp.float32),
    )(embed_t, W, b)


@functools.lru_cache(maxsize=None)
def _make_pool_kernel(B, L):
    n_rows = B // _NW
    n_idx = n_rows * L
    assert n_rows % _LANES == 0 and n_rows % 128 == 0
    mesh = plsc.VectorSubcoreMesh(core_axis_name="c", subcore_axis_name="s")

    @functools.partial(
        pl.kernel,
        out_type=jax.ShapeDtypeStruct((B,), jnp.float32),
        mesh=mesh,
        scratch_types=[
            pltpu.VMEM((L, n_rows), jnp.int32),
            pltpu.VMEM((n_idx,), jnp.float32),
            pltpu.VMEM((n_rows,), jnp.float32),
            pltpu.SemaphoreType.DMA,
        ],
    )
    def pool(xt_hbm, t_hbm, out_hbm, idx_v, vals_v, out_v, sem):
        cid = lax.axis_index("c")
        sid = lax.axis_index("s")
        wid = sid * _NC + cid
        off = wid * n_rows
        pltpu.sync_copy(xt_hbm.at[:, pl.ds(off, n_rows)], idx_v)
        gathers = [
            pltpu.async_copy(
                t_hbm.at[idx_v.at[c]],
                vals_v.at[pl.ds(c * n_rows, n_rows)],
                sem,
            )
            for c in range(L)
        ]
        for cp in gathers:
            cp.wait()
        inv_l = 1.0 / L
        for j in range(n_rows // _LANES):
            col = j * _LANES

            def body(l, acc):
                return acc + vals_v[pl.ds(l * n_rows + col, _LANES)]

            acc = lax.fori_loop(0, L, body, jnp.zeros((_LANES,), jnp.float32))
            out_v[pl.ds(col, _LANES)] = 1.0 / (1.0 + jnp.exp(acc * -inv_l))
        pltpu.sync_copy(out_v, out_hbm.at[pl.ds(wid * n_rows, n_rows)])

    return pool


def kernel(x, embed, W, b):
    B, L = x.shape
    t = _precompute_table(
        embed.T, W.astype(jnp.float32), b.astype(jnp.float32)
    )
    xt = x.astype(jnp.int32).T
    out = _make_pool_kernel(B, L)(xt, t)
    return out.reshape(B, 1)


# speedup vs baseline: 3.5000x; 1.1691x over previous
import functools

import jax
import jax.numpy as jnp
from jax import lax
from jax.experimental import pallas as pl
from jax.experimental.pallas import tpu as pltpu
from jax.experimental.pallas import tpu_sc as plsc

_NC = 2
_NS = 16
_NW = _NC * _NS
_LANES = 16
_TBC = 8192


def _matvec_body(et_ref, w_ref, b_ref, t_ref):
    r = jax.lax.dot_general(
        w_ref[...], et_ref[...],
        (((1,), (0,)), ((), ())),
        preferred_element_type=jnp.float32,
    )
    t_ref[...] = r[0] + b_ref[0]


def _precompute_table(embed_t, W, b):
    D, V = embed_t.shape
    return pl.pallas_call(
        _matvec_body,
        grid=(pl.cdiv(V, _TBC),),
        in_specs=[
            pl.BlockSpec((D, _TBC), lambda i: (0, i)),
            pl.BlockSpec((1, D), lambda i: (0, 0)),
            pl.BlockSpec(memory_space=pltpu.SMEM),
        ],
        out_specs=pl.BlockSpec((_TBC,), lambda i: (i,)),
        out_shape=jax.ShapeDtypeStruct((V,), jnp.float32),
    )(embed_t, W, b)


@functools.lru_cache(maxsize=None)
def _make_pool_kernel(B, L):
    n_rows = B // _NW
    n_idx = n_rows * L
    assert n_rows % _LANES == 0 and n_rows % 128 == 0
    mesh = plsc.VectorSubcoreMesh(core_axis_name="c", subcore_axis_name="s")

    @functools.partial(
        pl.kernel,
        out_type=jax.ShapeDtypeStruct((B,), jnp.float32),
        mesh=mesh,
        scratch_types=[
            pltpu.VMEM((L, n_rows), jnp.int32),
            pltpu.VMEM((n_idx,), jnp.float32),
            pltpu.VMEM((n_rows,), jnp.float32),
            pltpu.SemaphoreType.DMA,
        ],
    )
    def pool(xt_hbm, t_hbm, out_hbm, idx_v, vals_v, out_v, sem):
        cid = lax.axis_index("c")
        sid = lax.axis_index("s")
        wid = sid * _NC + cid
        off = wid * n_rows
        pltpu.sync_copy(xt_hbm.at[:, pl.ds(off, n_rows)], idx_v)
        gathers = [
            pltpu.async_copy(
                t_hbm.at[idx_v.at[c]],
                vals_v.at[pl.ds(c * n_rows, n_rows)],
                sem,
            )
            for c in range(L)
        ]
        n_acc = n_rows // _LANES
        accs = [jnp.zeros((_LANES,), jnp.float32) for _ in range(n_acc)]
        for c in range(L):
            gathers[c].wait()
            base = c * n_rows
            for j in range(n_acc):
                accs[j] = accs[j] + vals_v[pl.ds(base + j * _LANES, _LANES)]
        inv_l = 1.0 / L
        for j in range(n_acc):
            out_v[pl.ds(j * _LANES, _LANES)] = 1.0 / (
                1.0 + jnp.exp(accs[j] * -inv_l)
            )
        pltpu.sync_copy(out_v, out_hbm.at[pl.ds(wid * n_rows, n_rows)])

    return pool


def kernel(x, embed, W, b):
    B, L = x.shape
    t = _precompute_table(
        embed.T, W.astype(jnp.float32), b.astype(jnp.float32)
    )
    xt = x.astype(jnp.int32).T
    out = _make_pool_kernel(B, L)(xt, t)
    return out.reshape(B, 1)


# speedup vs baseline: 3.7949x; 1.0843x over previous
import functools

import jax
import jax.numpy as jnp
from jax import lax
from jax.experimental import pallas as pl
from jax.experimental.pallas import tpu as pltpu
from jax.experimental.pallas import tpu_sc as plsc

_NC = 2
_NS = 16
_NW = _NC * _NS
_LANES = 16
_TBC = 16384


def _matvec_body(et_ref, w_ref, b_ref, t_ref):
    r = jax.lax.dot_general(
        w_ref[...], et_ref[...],
        (((1,), (0,)), ((), ())),
        preferred_element_type=jnp.float32,
    )
    t_ref[...] = r[0] + b_ref[0]


def _precompute_table(embed_t, W, b):
    D, V = embed_t.shape
    return pl.pallas_call(
        _matvec_body,
        grid=(pl.cdiv(V, _TBC),),
        in_specs=[
            pl.BlockSpec((D, _TBC), lambda i: (0, i)),
            pl.BlockSpec((1, D), lambda i: (0, 0)),
            pl.BlockSpec(memory_space=pltpu.SMEM),
        ],
        out_specs=pl.BlockSpec((_TBC,), lambda i: (i,)),
        out_shape=jax.ShapeDtypeStruct((V,), jnp.float32),
    )(embed_t, W, b)


@functools.lru_cache(maxsize=None)
def _make_pool_kernel(B, L):
    n_rows = B // _NW
    n_idx = n_rows * L
    assert n_rows % _LANES == 0 and n_rows % 128 == 0
    mesh = plsc.VectorSubcoreMesh(core_axis_name="c", subcore_axis_name="s")

    @functools.partial(
        pl.kernel,
        out_type=jax.ShapeDtypeStruct((B,), jnp.float32),
        mesh=mesh,
        scratch_types=[
            pltpu.VMEM((L, n_rows), jnp.int32),
            pltpu.VMEM((n_idx,), jnp.float32),
            pltpu.VMEM((n_rows,), jnp.float32),
            pltpu.SemaphoreType.DMA,
        ],
    )
    def pool(xt_hbm, t_hbm, out_hbm, idx_v, vals_v, out_v, sem):
        cid = lax.axis_index("c")
        sid = lax.axis_index("s")
        wid = sid * _NC + cid
        off = wid * n_rows
        pltpu.sync_copy(xt_hbm.at[:, pl.ds(off, n_rows)], idx_v)
        gathers = [
            pltpu.async_copy(
                t_hbm.at[idx_v.at[c]],
                vals_v.at[pl.ds(c * n_rows, n_rows)],
                sem,
            )
            for c in range(L)
        ]
        n_acc = n_rows // _LANES
        accs = [jnp.zeros((_LANES,), jnp.float32) for _ in range(n_acc)]
        for c in range(L):
            gathers[c].wait()
            base = c * n_rows
            for j in range(n_acc):
                accs[j] = accs[j] + vals_v[pl.ds(base + j * _LANES, _LANES)]
        inv_l = 1.0 / L
        for j in range(n_acc):
            out_v[pl.ds(j * _LANES, _LANES)] = 1.0 / (
                1.0 + jnp.exp(accs[j] * -inv_l)
            )
        pltpu.sync_copy(out_v, out_hbm.at[pl.ds(wid * n_rows, n_rows)])

    return pool


def kernel(x, embed, W, b):
    B, L = x.shape
    t = _precompute_table(
        embed.T, W.astype(jnp.float32), b.astype(jnp.float32)
    )
    xt = x.astype(jnp.int32).T
    out = _make_pool_kernel(B, L)(xt, t)
    return out.reshape(B, 1)


# speedup vs baseline: 3.9385x; 1.0379x over previous
import functools

import jax
import jax.numpy as jnp
from jax import lax
from jax.experimental import pallas as pl
from jax.experimental.pallas import tpu as pltpu
from jax.experimental.pallas import tpu_sc as plsc

_NC = 2
_NS = 16
_NW = _NC * _NS
_LANES = 16
_TBC = 32768


def _matvec_body(et_ref, w_ref, b_ref, t_ref):
    r = jax.lax.dot_general(
        w_ref[...], et_ref[...],
        (((1,), (0,)), ((), ())),
        preferred_element_type=jnp.float32,
    )
    t_ref[...] = r[0] + b_ref[0]


def _precompute_table(embed_t, W, b):
    D, V = embed_t.shape
    return pl.pallas_call(
        _matvec_body,
        grid=(pl.cdiv(V, _TBC),),
        in_specs=[
            pl.BlockSpec((D, _TBC), lambda i: (0, i)),
            pl.BlockSpec((1, D), lambda i: (0, 0)),
            pl.BlockSpec(memory_space=pltpu.SMEM),
        ],
        out_specs=pl.BlockSpec((_TBC,), lambda i: (i,)),
        out_shape=jax.ShapeDtypeStruct((V,), jnp.float32),
    )(embed_t, W, b)


@functools.lru_cache(maxsize=None)
def _make_pool_kernel(B, L):
    n_rows = B // _NW
    n_idx = n_rows * L
    assert n_rows % _LANES == 0 and n_rows % 128 == 0
    mesh = plsc.VectorSubcoreMesh(core_axis_name="c", subcore_axis_name="s")

    @functools.partial(
        pl.kernel,
        out_type=jax.ShapeDtypeStruct((B,), jnp.float32),
        mesh=mesh,
        scratch_types=[
            pltpu.VMEM((L, n_rows), jnp.int32),
            pltpu.VMEM((n_idx,), jnp.float32),
            pltpu.VMEM((n_rows,), jnp.float32),
            pltpu.SemaphoreType.DMA,
        ],
    )
    def pool(xt_hbm, t_hbm, out_hbm, idx_v, vals_v, out_v, sem):
        cid = lax.axis_index("c")
        sid = lax.axis_index("s")
        wid = sid * _NC + cid
        off = wid * n_rows
        pltpu.sync_copy(xt_hbm.at[:, pl.ds(off, n_rows)], idx_v)
        gathers = [
            pltpu.async_copy(
                t_hbm.at[idx_v.at[c]],
                vals_v.at[pl.ds(c * n_rows, n_rows)],
                sem,
            )
            for c in range(L)
        ]
        n_acc = n_rows // _LANES
        accs = [jnp.zeros((_LANES,), jnp.float32) for _ in range(n_acc)]
        for c in range(L):
            gathers[c].wait()
            base = c * n_rows
            for j in range(n_acc):
                accs[j] = accs[j] + vals_v[pl.ds(base + j * _LANES, _LANES)]
        inv_l = 1.0 / L
        for j in range(n_acc):
            out_v[pl.ds(j * _LANES, _LANES)] = 1.0 / (
                1.0 + jnp.exp(accs[j] * -inv_l)
            )
        pltpu.sync_copy(out_v, out_hbm.at[pl.ds(wid * n_rows, n_rows)])

    return pool


def kernel(x, embed, W, b):
    B, L = x.shape
    t = _precompute_table(
        embed.T, W.astype(jnp.float32), b.astype(jnp.float32)
    )
    xt = x.astype(jnp.int32).T
    out = _make_pool_kernel(B, L)(xt, t)
    return out.reshape(B, 1)
